# baseline (speedup 1.0000x reference)
import functools

import jax
import jax.numpy as jnp
from jax.experimental import pallas as pl
from jax.experimental.pallas import tpu as pltpu

_NUM_CLASSES = 40
_LANES = 128


def _fused_kernel(n_inst, n_frames, fp_ref, ip_ref, px_ref, py_ref,
                  ml_ref, mg_ref, md_ref, fl_s, fv_s):
    n_half = n_frames // 2
    j_idx = pl.program_id(1)
    jm = jnp.minimum(j_idx, n_half - 1)

    px = px_ref[...]
    py = py_ref[...]
    rows, lanes = px.shape
    idx = (jax.lax.broadcasted_iota(jnp.int32, (rows, lanes), 0) * lanes
           + jax.lax.broadcasted_iota(jnp.int32, (rows, lanes), 1))
    big_idx = jnp.int32(rows * lanes)
    nc = md_ref.shape[-1]
    ci = jax.lax.broadcasted_iota(jnp.int32, (1, nc), 1)

    for f in range(2):
        flT = fl_s[f].T
        fvT = fv_s[f].T
        for r in range(rows):
            lbl = flT[:, r:r + 1]
            v = fvT[:, r:r + 1]
            md_ref[0, f, r * lanes:(r + 1) * lanes, :] = jnp.where(
                lbl == ci, v, 0.0)

    st = []
    for f in range(2):
        st.append(dict(
            claimed=jnp.zeros((rows, lanes), dtype=jnp.bool_),
            killed=jnp.zeros((rows, lanes), dtype=jnp.bool_),
            ml=jnp.full((rows, lanes), -1, dtype=jnp.int32),
            mg=jnp.full((rows, lanes), -1, dtype=jnp.int32),
            fl=jnp.full((rows, lanes), -1, dtype=jnp.int32),
            fv=jnp.zeros((rows, lanes), dtype=jnp.float32),
        ))

    for n in range(n_inst):
        for f in range(2):
            s = st[f]
            cx = fp_ref[0, f, 0, n]
            cy = fp_ref[0, f, 1, n]
            w = fp_ref[0, f, 2, n]
            h = fp_ref[0, f, 3, n]
            lab = ip_ref[0, f, 0, n]
            gid = ip_ref[0, f, 1, n]
            act = ip_ref[0, f, 2, n]

            dx = (cx - px) / jnp.maximum(w, 0.05)
            dy = (cy - py) / jnp.maximum(h, 0.05)
            d = dx * dx + dy * dy
            d_eff = jnp.where(s["claimed"], 1e9, d)

            inner = d_eff < 0.5
            minv = jnp.min(d_eff)
            any_inner = minv < 0.5
            min_idx = jnp.min(jnp.where(d_eff == minv, idx, big_idx))
            fallback = idx == min_idx

            pos = ((inner & any_inner)
                   | (fallback & jnp.logical_not(any_inner))) & (act != 0)
            val = 1.0 - 2.0 * jnp.clip(d_eff, 0.0, 0.5)

            new_first = pos & jnp.logical_not(s["claimed"])
            reclaim = pos & s["claimed"]
            s["fl"] = jnp.where(new_first, lab, s["fl"])
            s["fv"] = jnp.where(new_first, val, s["fv"])
            s["killed"] = s["killed"] | (reclaim & (s["fl"] == lab))
            s["ml"] = jnp.where(pos, lab, s["ml"])
            s["mg"] = jnp.where(pos, gid, s["mg"])
            s["claimed"] = s["claimed"] | pos

    t_base = 2 * jm
    for f in range(2):
        s = st[f]
        alive = s["claimed"] & jnp.logical_not(s["killed"])
        ml_ref[0, pl.ds(t_base + f, 1), :] = s["ml"].reshape(1, rows * lanes)
        mg_ref[0, pl.ds(t_base + f, 1), :] = s["mg"].reshape(1, rows * lanes)
        fl_s[f] = jnp.where(alive, s["fl"], -1)
        fv_s[f] = jnp.where(alive, s["fv"], 0.0)


def kernel(gt_boxes, gt_labels, gt_ids, ref_points, spatial_shapes):
    B, N, T, _ = gt_boxes.shape
    P = ref_points.shape[0]
    C = _NUM_CLASSES
    L = _LANES
    R = P // L
    TH = T // 2

    x0, y0, x1, y1 = (gt_boxes[..., 0], gt_boxes[..., 1],
                      gt_boxes[..., 2], gt_boxes[..., 3])
    cx = (x0 + x1) * 0.5
    cy = (y0 + y1) * 0.5
    w = x1 - x0
    h = y1 - y0
    area = (w * h).mean(-1)
    order = jnp.argsort(area, axis=-1)
    bidx = jnp.arange(B)[:, None]

    cx_s = cx[bidx, order]
    cy_s = cy[bidx, order]
    w_s = w[bidx, order]
    h_s = h[bidx, order]
    labels_s = gt_labels[bidx, order]
    ids_s = gt_ids[bidx, order]
    valid = ((w_s > 0.0) & (h_s > 0.0)).any(-1) & (labels_s >= 0)
    active = valid[:, :, None] & (ids_s != -1)

    fp = jnp.zeros((B, T, 8, L), jnp.float32)
    fp = fp.at[:, :, 0, :N].set(cx_s.transpose(0, 2, 1))
    fp = fp.at[:, :, 1, :N].set(cy_s.transpose(0, 2, 1))
    fp = fp.at[:, :, 2, :N].set(w_s.transpose(0, 2, 1))
    fp = fp.at[:, :, 3, :N].set(h_s.transpose(0, 2, 1))

    ip = jnp.zeros((B, T, 8, L), jnp.int32)
    ip = ip.at[:, :, 0, :N].set(jnp.broadcast_to(labels_s[:, None, :], (B, T, N)))
    ip = ip.at[:, :, 1, :N].set(ids_s.transpose(0, 2, 1))
    ip = ip.at[:, :, 2, :N].set(active.transpose(0, 2, 1).astype(jnp.int32))

    px2 = ref_points[:, 0].reshape(R, L)
    py2 = ref_points[:, 1].reshape(R, L)

    ml, mg, md = pl.pallas_call(
        functools.partial(_fused_kernel, N, T),
        grid=(B, TH + 1),
        in_specs=[
            pl.BlockSpec((1, 2, 8, L),
                         lambda b, j: (b, jnp.minimum(j, TH - 1), 0, 0)),
            pl.BlockSpec((1, 2, 8, L),
                         lambda b, j: (b, jnp.minimum(j, TH - 1), 0, 0)),
            pl.BlockSpec((R, L), lambda b, j: (0, 0)),
            pl.BlockSpec((R, L), lambda b, j: (0, 0)),
        ],
        out_specs=[
            pl.BlockSpec((1, T, P), lambda b, j: (b, 0, 0)),
            pl.BlockSpec((1, T, P), lambda b, j: (b, 0, 0)),
            pl.BlockSpec((1, 2, P, C),
                         lambda b, j: (b, jnp.maximum(j - 1, 0), 0, 0)),
        ],
        out_shape=[
            jax.ShapeDtypeStruct((B, T, P), jnp.int32),
            jax.ShapeDtypeStruct((B, T, P), jnp.int32),
            jax.ShapeDtypeStruct((B, T, P, C), jnp.float32),
        ],
        scratch_shapes=[
            pltpu.VMEM((2, R, L), jnp.int32),
            pltpu.VMEM((2, R, L), jnp.float32),
        ],
    )(fp, ip, px2, py2)

    return (ml, md, mg)


# speedup vs baseline: 1.0407x; 1.0407x over previous
import functools

import jax
import jax.numpy as jnp
from jax.experimental import pallas as pl

_NUM_CLASSES = 40
_LANES = 128


def _fused_kernel(n_inst, fp_ref, ip_ref, px_ref, py_ref,
                  ml_ref, mg_ref, md_ref):
    px = px_ref[...]
    py = py_ref[...]
    rows, lanes = px.shape
    idx = (jax.lax.broadcasted_iota(jnp.int32, (rows, lanes), 0) * lanes
           + jax.lax.broadcasted_iota(jnp.int32, (rows, lanes), 1))
    big_idx = jnp.int32(rows * lanes)

    claimed = jnp.zeros((rows, lanes), dtype=jnp.bool_)
    killed = jnp.zeros((rows, lanes), dtype=jnp.bool_)
    ml = jnp.full((rows, lanes), -1, dtype=jnp.int32)
    mg = jnp.full((rows, lanes), -1, dtype=jnp.int32)
    fl = jnp.full((rows, lanes), -1, dtype=jnp.int32)
    fv = jnp.zeros((rows, lanes), dtype=jnp.float32)

    for n in range(n_inst):
        cx = fp_ref[0, 0, n]
        cy = fp_ref[0, 1, n]
        w = fp_ref[0, 2, n]
        h = fp_ref[0, 3, n]
        lab = ip_ref[0, 0, n]
        gid = ip_ref[0, 1, n]
        act = ip_ref[0, 2, n]

        dx = (cx - px) / jnp.maximum(w, 0.05)
        dy = (cy - py) / jnp.maximum(h, 0.05)
        d = dx * dx + dy * dy
        d_eff = jnp.where(claimed, 1e9, d)

        inner = d_eff < 0.5
        minv = jnp.min(d_eff)
        any_inner = minv < 0.5
        min_idx = jnp.min(jnp.where(d_eff == minv, idx, big_idx))
        fallback = idx == min_idx

        pos = ((inner & any_inner)
               | (fallback & jnp.logical_not(any_inner))) & (act != 0)
        val = 1.0 - 2.0 * jnp.clip(d_eff, 0.0, 0.5)

        new_first = pos & jnp.logical_not(claimed)
        reclaim = pos & claimed
        fl = jnp.where(new_first, lab, fl)
        fv = jnp.where(new_first, val, fv)
        killed = killed | (reclaim & (fl == lab))
        ml = jnp.where(pos, lab, ml)
        mg = jnp.where(pos, gid, mg)
        claimed = claimed | pos

    alive = claimed & jnp.logical_not(killed)
    ml_ref[0] = ml
    mg_ref[0] = mg

    nc = md_ref.shape[-1]
    fl_dead = jnp.where(alive, fl, -1)
    fv_dead = jnp.where(alive, fv, 0.0)
    flT = fl_dead.T
    fvT = fv_dead.T
    ci = jax.lax.broadcasted_iota(jnp.int32, (1, nc), 1)
    for r in range(rows):
        lbl = flT[:, r:r + 1]
        v = fvT[:, r:r + 1]
        md_ref[0, r * lanes:(r + 1) * lanes, :] = jnp.where(lbl == ci, v, 0.0)


def kernel(gt_boxes, gt_labels, gt_ids, ref_points, spatial_shapes):
    B, N, T, _ = gt_boxes.shape
    P = ref_points.shape[0]
    C = _NUM_CLASSES
    L = _LANES
    R = P // L

    x0, y0, x1, y1 = (gt_boxes[..., 0], gt_boxes[..., 1],
                      gt_boxes[..., 2], gt_boxes[..., 3])
    cx = (x0 + x1) * 0.5
    cy = (y0 + y1) * 0.5
    w = x1 - x0
    h = y1 - y0
    area = (w * h).mean(-1)
    order = jnp.argsort(area, axis=-1)
    bidx = jnp.arange(B)[:, None]

    cx_s = cx[bidx, order]
    cy_s = cy[bidx, order]
    w_s = w[bidx, order]
    h_s = h[bidx, order]
    labels_s = gt_labels[bidx, order]
    ids_s = gt_ids[bidx, order]
    valid = ((w_s > 0.0) & (h_s > 0.0)).any(-1) & (labels_s >= 0)
    active = valid[:, :, None] & (ids_s != -1)

    fp = jnp.zeros((B, T, 8, L), jnp.float32)
    fp = fp.at[:, :, 0, :N].set(cx_s.transpose(0, 2, 1))
    fp = fp.at[:, :, 1, :N].set(cy_s.transpose(0, 2, 1))
    fp = fp.at[:, :, 2, :N].set(w_s.transpose(0, 2, 1))
    fp = fp.at[:, :, 3, :N].set(h_s.transpose(0, 2, 1))
    fp = fp.reshape(B * T, 8, L)

    ip = jnp.zeros((B, T, 8, L), jnp.int32)
    ip = ip.at[:, :, 0, :N].set(jnp.broadcast_to(labels_s[:, None, :], (B, T, N)))
    ip = ip.at[:, :, 1, :N].set(ids_s.transpose(0, 2, 1))
    ip = ip.at[:, :, 2, :N].set(active.transpose(0, 2, 1).astype(jnp.int32))
    ip = ip.reshape(B * T, 8, L)

    px2 = ref_points[:, 0].reshape(R, L)
    py2 = ref_points[:, 1].reshape(R, L)

    BT = B * T
    ml16, mg16, md = pl.pallas_call(
        functools.partial(_fused_kernel, N),
        grid=(BT,),
        in_specs=[
            pl.BlockSpec((1, 8, L), lambda i: (i, 0, 0)),
            pl.BlockSpec((1, 8, L), lambda i: (i, 0, 0)),
            pl.BlockSpec((R, L), lambda i: (0, 0)),
            pl.BlockSpec((R, L), lambda i: (0, 0)),
        ],
        out_specs=[
            pl.BlockSpec((1, R, L), lambda i: (i, 0, 0)),
            pl.BlockSpec((1, R, L), lambda i: (i, 0, 0)),
            pl.BlockSpec((1, P, C), lambda i: (i, 0, 0)),
        ],
        out_shape=[
            jax.ShapeDtypeStruct((BT, R, L), jnp.int32),
            jax.ShapeDtypeStruct((BT, R, L), jnp.int32),
            jax.ShapeDtypeStruct((BT, P, C), jnp.float32),
        ],
    )(fp, ip, px2, py2)

    ml = ml16.reshape(B, T, P)
    mg = mg16.reshape(B, T, P)
    md = md.reshape(B, T, P, C)
    return (ml, md, mg)
